# baseline (speedup 1.0000x reference)
import functools

import jax
import jax.numpy as jnp
from jax import lax
from jax.experimental import pallas as pl
from jax.experimental.pallas import tpu as pltpu
from jax.experimental.pallas import tpu_sc as plsc

N_TOKENS = 8192
D_MODEL = 768
PATHS = 8
NC = 2
NS = 16
NW = NC * NS
ROWS_PER_W = N_TOKENS // NW
CBLK = 64
NCOPY = ROWS_PER_W // CBLK
ZBLOCK = 2048


def _sc_copy_body(in_hbm, out0, b0, b1, gsem, s0, s1):
    cbufs = (b0, b1)
    ssems = (s0, s1)
    wid = lax.axis_index("s") * NC + lax.axis_index("c")
    base = wid * ROWS_PER_W
    scatters = [None, None]
    for j in range(NCOPY):
        lo = base + j * CBLK
        b = j % 2
        if scatters[b] is not None:
            scatters[b].wait()
        pltpu.async_copy(in_hbm.at[pl.ds(lo, CBLK)], cbufs[b], gsem).wait()
        scatters[b] = pltpu.async_copy(
            cbufs[b], out0.at[pl.ds(lo, CBLK)], ssems[b]
        )
    for c in scatters:
        if c is not None:
            c.wait()


_sc_copy = functools.partial(
    pl.kernel,
    mesh=plsc.VectorSubcoreMesh(core_axis_name="c", subcore_axis_name="s"),
    out_type=jax.ShapeDtypeStruct((N_TOKENS, D_MODEL), jnp.float32),
    scratch_types=[
        pltpu.VMEM((CBLK, D_MODEL), jnp.float32),
        pltpu.VMEM((CBLK, D_MODEL), jnp.float32),
        pltpu.SemaphoreType.DMA,
        pltpu.SemaphoreType.DMA,
        pltpu.SemaphoreType.DMA,
    ],
)(_sc_copy_body)


def _tc_zero_body(*out_refs):
    for r in out_refs:
        r[...] = jnp.zeros_like(r)


def _tc_zeros(n, d, dtype, count):
    spec = pl.BlockSpec((ZBLOCK, d), lambda i: (i, 0))
    return pl.pallas_call(
        _tc_zero_body,
        grid=(n // ZBLOCK,),
        in_specs=[],
        out_specs=tuple(spec for _ in range(count)),
        out_shape=tuple(
            jax.ShapeDtypeStruct((n, d), dtype) for _ in range(count)
        ),
    )()


def kernel(inputs):
    n, d = inputs.shape
    out0 = _sc_copy(inputs)
    za = _tc_zeros(n, d, inputs.dtype, 4)
    zb = _tc_zeros(n, d, inputs.dtype, 3)
    return (out0,) + tuple(za) + tuple(zb)


# speedup vs baseline: 1.0627x; 1.0627x over previous
import functools

import jax
import jax.numpy as jnp
from jax import lax
from jax.experimental import pallas as pl
from jax.experimental.pallas import tpu as pltpu
from jax.experimental.pallas import tpu_sc as plsc

N_TOKENS = 8192
D_MODEL = 768
PATHS = 8
NC = 2
NS = 16
NW = NC * NS
ROWS_PER_W = N_TOKENS // NW
CBLK = 64
NCOPY = ROWS_PER_W // CBLK
ZBLOCK = 512


def _sc_copy_body(in_hbm, out0, b0, b1, gsem, s0, s1):
    cbufs = (b0, b1)
    ssems = (s0, s1)
    wid = lax.axis_index("s") * NC + lax.axis_index("c")
    base = wid * ROWS_PER_W
    scatters = [None, None]
    for j in range(NCOPY):
        lo = base + j * CBLK
        b = j % 2
        if scatters[b] is not None:
            scatters[b].wait()
        pltpu.async_copy(in_hbm.at[pl.ds(lo, CBLK)], cbufs[b], gsem).wait()
        scatters[b] = pltpu.async_copy(
            cbufs[b], out0.at[pl.ds(lo, CBLK)], ssems[b]
        )
    for c in scatters:
        if c is not None:
            c.wait()


_sc_copy = functools.partial(
    pl.kernel,
    mesh=plsc.VectorSubcoreMesh(core_axis_name="c", subcore_axis_name="s"),
    out_type=jax.ShapeDtypeStruct((N_TOKENS, D_MODEL), jnp.float32),
    scratch_types=[
        pltpu.VMEM((CBLK, D_MODEL), jnp.float32),
        pltpu.VMEM((CBLK, D_MODEL), jnp.float32),
        pltpu.SemaphoreType.DMA,
        pltpu.SemaphoreType.DMA,
        pltpu.SemaphoreType.DMA,
    ],
)(_sc_copy_body)


def _tc_zero_body(*out_refs):
    for r in out_refs:
        r[...] = jnp.zeros_like(r)


def _tc_zeros(n, d, dtype, count):
    spec = pl.BlockSpec((ZBLOCK, d), lambda i: (i, 0))
    return pl.pallas_call(
        _tc_zero_body,
        grid=(n // ZBLOCK,),
        in_specs=[],
        out_specs=tuple(spec for _ in range(count)),
        out_shape=tuple(
            jax.ShapeDtypeStruct((n, d), dtype) for _ in range(count)
        ),
    )()


def kernel(inputs):
    n, d = inputs.shape
    out0 = _sc_copy(inputs)
    zeros = _tc_zeros(n, d, inputs.dtype, PATHS - 1)
    return (out0,) + tuple(zeros)
